# Initial kernel scaffold; baseline (speedup 1.0000x reference)
import jax
import jax.numpy as jnp
from jax.experimental import pallas as pl


def kernel(ctrl_pts, uspan, Nu):
    raise NotImplementedError("write your pallas kernel here")


# speedup vs baseline: 1.5772x; 1.5772x over previous
import functools

import jax
import jax.numpy as jnp
from jax import lax
from jax.experimental import pallas as pl
from jax.experimental.pallas import tpu as pltpu
from jax.experimental.pallas import tpu_sc as plsc

_BATCH = 1024
_M = 64
_P = 3
_DIM = 3
_OUT_DIM = 1024

_NC = 2
_NS = 16
_L = 16
_NW = _NC * _NS
_B_PER_W = _BATCH // _NW

_CP_ROW = _M * (_DIM + 1)
_OUT_ROW = _OUT_DIM * _DIM
_N_CHUNK = _OUT_DIM // _L


def _sc_body(ctrl_hbm, uspan_hbm, nut_hbm, out_hbm, ctrl_v, uspan_v, nut_v,
             out_v):
    wid = lax.axis_index("s") * _NC + lax.axis_index("c")
    base = wid * _B_PER_W

    pltpu.sync_copy(ctrl_hbm.at[pl.ds(base * _CP_ROW, _B_PER_W * _CP_ROW)],
                    ctrl_v)
    pltpu.sync_copy(uspan_hbm, uspan_v)
    pltpu.sync_copy(nut_hbm, nut_v)

    lane = lax.iota(jnp.int32, _L)

    def b_body(b, carry):
        cbase = b * _CP_ROW

        def c_body(c, carry2):
            u = uspan_v[pl.ds(c * _L, _L)]
            denom = jnp.zeros((_L,), jnp.float32)
            num0 = jnp.zeros((_L,), jnp.float32)
            num1 = jnp.zeros((_L,), jnp.float32)
            num2 = jnp.zeros((_L,), jnp.float32)
            for j in range(_P + 1):
                fj = cbase + (u + (j - _P)) * (_DIM + 1)
                w = plsc.load_gather(ctrl_v, [fj + _DIM])
                nu = nut_v[j, pl.ds(c * _L, _L)]
                a = w * nu
                denom = denom + a
                num0 = num0 + a * plsc.load_gather(ctrl_v, [fj])
                num1 = num1 + a * plsc.load_gather(ctrl_v, [fj + 1])
                num2 = num2 + a * plsc.load_gather(ctrl_v, [fj + 2])
            inv = 1.0 / denom
            sidx = c * (_L * _DIM) + lane * _DIM
            plsc.store_scatter(out_v, [sidx], num0 * inv)
            plsc.store_scatter(out_v, [sidx + 1], num1 * inv)
            plsc.store_scatter(out_v, [sidx + 2], num2 * inv)
            return carry2

        lax.fori_loop(0, _N_CHUNK, c_body, 0, unroll=False)
        pltpu.sync_copy(out_v, out_hbm.at[base + b])
        return carry

    lax.fori_loop(0, _B_PER_W, b_body, 0, unroll=False)


@jax.jit
def _sc_eval(ctrl_flat, uspan, nut):
    mesh = plsc.VectorSubcoreMesh(core_axis_name="c", subcore_axis_name="s",
                                  num_cores=_NC, num_subcores=_NS)
    f = pl.kernel(
        _sc_body,
        out_type=jax.ShapeDtypeStruct((_BATCH, _OUT_ROW), jnp.float32),
        mesh=mesh,
        scratch_types=[
            pltpu.VMEM((_B_PER_W * _CP_ROW,), jnp.float32),
            pltpu.VMEM((_OUT_DIM,), jnp.int32),
            pltpu.VMEM((_P + 1, _OUT_DIM), jnp.float32),
            pltpu.VMEM((_OUT_ROW,), jnp.float32),
        ],
        compiler_params=pltpu.CompilerParams(needs_layout_passes=False),
    )
    return f(ctrl_flat, uspan, nut)


def kernel(ctrl_pts, uspan, Nu):
    ctrl_flat = ctrl_pts.reshape(_BATCH * _CP_ROW)
    nut = Nu.T
    out = _sc_eval(ctrl_flat, uspan, nut)
    return out.reshape(_BATCH, _OUT_DIM, _DIM)


# speedup vs baseline: 1.8951x; 1.2015x over previous
import functools

import jax
import jax.numpy as jnp
from jax import lax
from jax.experimental import pallas as pl
from jax.experimental.pallas import tpu as pltpu
from jax.experimental.pallas import tpu_sc as plsc

_BATCH = 1024
_M = 64
_P = 3
_DIM = 3
_OUT_DIM = 1024

_NC = 2
_NS = 16
_L = 16
_NW = _NC * _NS
_B_PER_W = _BATCH // _NW

_CP_ROW = _M * (_DIM + 1)
_OUT_ROW = _OUT_DIM * _DIM
_N_CHUNK = _OUT_DIM // _L


def _sc_body(ctrl_hbm, uspan_hbm, nut_hbm, out_hbm, ctrl_v, uspan_v, nut_v,
             idx_v, out_v, sem):
    wid = lax.axis_index("s") * _NC + lax.axis_index("c")
    base = wid * _B_PER_W

    pltpu.sync_copy(ctrl_hbm.at[pl.ds(base * _CP_ROW, _B_PER_W * _CP_ROW)],
                    ctrl_v)
    pltpu.sync_copy(uspan_hbm, uspan_v)
    pltpu.sync_copy(nut_hbm, nut_v)

    lane3 = lax.iota(jnp.int32, _L) * _DIM

    @plsc.parallel_loop(0, _N_CHUNK)
    def _pre(c):
        u = uspan_v[pl.ds(c * _L, _L)]
        for j in range(_P + 1):
            idx_v[pl.ds((c * (_P + 1) + j) * _L, _L)] = (
                (u + (j - _P)) * (_DIM + 1))

    def run_chunks(c_lo, c_hi):
        def c_body(c, carry):
            fj = [idx_v[pl.ds((c * (_P + 1) + j) * _L, _L)]
                  for j in range(_P + 1)]
            nu = [nut_v[j, pl.ds(c * _L, _L)] for j in range(_P + 1)]
            sidx = lane3 + c * (_L * _DIM)

            @plsc.parallel_loop(0, _B_PER_W, unroll=2)
            def _bloop(b):
                cbase = b * _CP_ROW
                g0 = fj[0] + cbase
                w = plsc.load_gather(ctrl_v, [g0 + _DIM])
                a = nu[0] * w
                denom = a
                num0 = a * plsc.load_gather(ctrl_v, [g0])
                num1 = a * plsc.load_gather(ctrl_v, [g0 + 1])
                num2 = a * plsc.load_gather(ctrl_v, [g0 + 2])
                for j in range(1, _P + 1):
                    gj = fj[j] + cbase
                    w = plsc.load_gather(ctrl_v, [gj + _DIM])
                    a = nu[j] * w
                    denom = denom + a
                    num0 = num0 + a * plsc.load_gather(ctrl_v, [gj])
                    num1 = num1 + a * plsc.load_gather(ctrl_v, [gj + 1])
                    num2 = num2 + a * plsc.load_gather(ctrl_v, [gj + 2])
                inv = 1.0 / denom
                bvec = jnp.full((_L,), b, jnp.int32)
                plsc.store_scatter(out_v, [bvec, sidx], num0 * inv)
                plsc.store_scatter(out_v, [bvec, sidx + 1], num1 * inv)
                plsc.store_scatter(out_v, [bvec, sidx + 2], num2 * inv)

            return carry

        lax.fori_loop(c_lo, c_hi, c_body, 0, unroll=False)

    half_cols = _OUT_ROW // 2
    run_chunks(0, _N_CHUNK // 2)
    cp1 = pltpu.async_copy(
        out_v.at[:, pl.ds(0, half_cols)],
        out_hbm.at[pl.ds(base, _B_PER_W), pl.ds(0, half_cols)], sem)
    run_chunks(_N_CHUNK // 2, _N_CHUNK)
    cp2 = pltpu.async_copy(
        out_v.at[:, pl.ds(half_cols, half_cols)],
        out_hbm.at[pl.ds(base, _B_PER_W), pl.ds(half_cols, half_cols)], sem)
    cp1.wait()
    cp2.wait()


@jax.jit
def _sc_eval(ctrl_flat, uspan, nut):
    mesh = plsc.VectorSubcoreMesh(core_axis_name="c", subcore_axis_name="s",
                                  num_cores=_NC, num_subcores=_NS)
    f = pl.kernel(
        _sc_body,
        out_type=jax.ShapeDtypeStruct((_BATCH, _OUT_ROW), jnp.float32),
        mesh=mesh,
        scratch_types=[
            pltpu.VMEM((_B_PER_W * _CP_ROW,), jnp.float32),
            pltpu.VMEM((_OUT_DIM,), jnp.int32),
            pltpu.VMEM((_P + 1, _OUT_DIM), jnp.float32),
            pltpu.VMEM(((_P + 1) * _OUT_DIM,), jnp.int32),
            pltpu.VMEM((_B_PER_W, _OUT_ROW), jnp.float32),
            pltpu.SemaphoreType.DMA,
        ],
        compiler_params=pltpu.CompilerParams(needs_layout_passes=False),
    )
    return f(ctrl_flat, uspan, nut)


def kernel(ctrl_pts, uspan, Nu):
    ctrl_flat = ctrl_pts.reshape(_BATCH * _CP_ROW)
    nut = Nu.T
    out = _sc_eval(ctrl_flat, uspan, nut)
    return out.reshape(_BATCH, _OUT_DIM, _DIM)
